# baseline (speedup 1.0000x reference)
import functools

import jax
import jax.numpy as jnp
from jax import lax
from jax.experimental import pallas as pl
from jax.experimental.pallas import tpu as pltpu
from jax.experimental.pallas import tpu_sc as plsc

_TABLE_ROWS = 8192
_D = 1024
_B_TOTAL = 8192
_NC = 2
_NS = 16
_NW = _NC * _NS
_B_PER_W = _B_TOTAL // _NW
_CHUNK = 32
_N_CHUNKS = _B_PER_W // _CHUNK

_SR = 4
_DS = _D * _SR
_BS_PER_W = _B_PER_W // _SR
_SCHUNK = 16
_SN_CHUNKS = _BS_PER_W // _SCHUNK


def _gather_sc_super(table, idx):
  mesh = plsc.VectorSubcoreMesh(core_axis_name="c", subcore_axis_name="s")

  @functools.partial(
      pl.kernel,
      mesh=mesh,
      out_type=jax.ShapeDtypeStruct((_NW, _SN_CHUNKS, _SCHUNK, _DS),
                                    jnp.float32),
      scratch_types=[
          pltpu.VMEM((_SN_CHUNKS, _SCHUNK), jnp.int32),
          pltpu.VMEM((_SCHUNK, _DS), jnp.float32),
          pltpu.SemaphoreType.DMA,
      ],
  )
  def k(table_hbm, idx_hbm, out_hbm, idx_v, buf, sem):
    wid = lax.axis_index("s") * _NC + lax.axis_index("c")
    pltpu.sync_copy(idx_hbm.at[wid], idx_v)
    for c in range(_SN_CHUNKS):
      pltpu.async_copy(table_hbm.at[idx_v.at[c]], buf, sem).wait()
      pltpu.sync_copy(buf, out_hbm.at[wid, c])

  return k(table, idx)


def _gather_sc(table, idx):
  mesh = plsc.VectorSubcoreMesh(core_axis_name="c", subcore_axis_name="s")

  @functools.partial(
      pl.kernel,
      mesh=mesh,
      out_type=jax.ShapeDtypeStruct((_NW, _N_CHUNKS, _CHUNK, _D),
                                    jnp.float32),
      scratch_types=[
          pltpu.VMEM((_N_CHUNKS, _CHUNK), jnp.int32),
          pltpu.VMEM((_CHUNK, _D), jnp.float32),
          pltpu.VMEM((_CHUNK, _D), jnp.float32),
          pltpu.SemaphoreType.DMA,
          pltpu.SemaphoreType.DMA,
          pltpu.SemaphoreType.DMA,
          pltpu.SemaphoreType.DMA,
      ],
  )
  def k(table_hbm, idx_hbm, out_hbm, idx_v, buf0, buf1,
        sg0, sg1, sw0, sw1):
    wid = lax.axis_index("s") * _NC + lax.axis_index("c")
    pltpu.sync_copy(idx_hbm.at[wid], idx_v)
    bufs, sgs, sws = (buf0, buf1), (sg0, sg1), (sw0, sw1)
    gathers = [None] * _N_CHUNKS
    writes = [None] * _N_CHUNKS
    gathers[0] = pltpu.async_copy(table_hbm.at[idx_v.at[0]], bufs[0], sgs[0])
    for c in range(_N_CHUNKS):
      gathers[c].wait()
      writes[c] = pltpu.async_copy(bufs[c % 2], out_hbm.at[wid, c],
                                   sws[c % 2])
      if c + 1 < _N_CHUNKS:
        if c >= 1:
          writes[c - 1].wait()
        gathers[c + 1] = pltpu.async_copy(
            table_hbm.at[idx_v.at[c + 1]], bufs[(c + 1) % 2], sgs[(c + 1) % 2])
    writes[_N_CHUNKS - 1].wait()

  return k(table, idx)


def kernel(position_ids, pos_emb):
  batch, seq = position_ids.shape
  table = pos_emb.reshape(_TABLE_ROWS // _SR, _DS)
  sidx = position_ids.reshape(-1)[::_SR] // _SR
  sidx = sidx.reshape(_NW, _SN_CHUNKS, _SCHUNK).astype(jnp.int32)
  out = _gather_sc_super(table, sidx)
  return out.reshape(1, batch, seq, _D)


# speedup vs baseline: 2.7533x; 2.7533x over previous
import functools

import jax
import jax.numpy as jnp
from jax import lax
from jax.experimental import pallas as pl
from jax.experimental.pallas import tpu as pltpu
from jax.experimental.pallas import tpu_sc as plsc

_TABLE_ROWS = 8192
_D = 1024
_B_TOTAL = 8192
_NC = 2
_NS = 16
_NW = _NC * _NS
_B_PER_W = _B_TOTAL // _NW
_CHUNK = 16
_N_CHUNKS = _B_PER_W // _CHUNK
_NBUF = 4


def _gather_sc(table, idx):
  mesh = plsc.VectorSubcoreMesh(core_axis_name="c", subcore_axis_name="s")

  @functools.partial(
      pl.kernel,
      mesh=mesh,
      out_type=jax.ShapeDtypeStruct((_NW, _N_CHUNKS, _CHUNK, _D),
                                    jnp.float32),
      scratch_types=[
          pltpu.VMEM((_N_CHUNKS, _CHUNK), jnp.int32),
      ] + [pltpu.VMEM((_CHUNK, _D), jnp.float32) for _ in range(_NBUF)]
        + [pltpu.SemaphoreType.DMA for _ in range(2 * _NBUF)],
  )
  def k(table_hbm, idx_hbm, out_hbm, idx_v, *rest):
    bufs = rest[:_NBUF]
    sgs = rest[_NBUF:2 * _NBUF]
    sws = rest[2 * _NBUF:]
    wid = lax.axis_index("s") * _NC + lax.axis_index("c")
    pltpu.sync_copy(idx_hbm.at[wid], idx_v)

    gathers = [None] * _N_CHUNKS
    writes = [None] * _N_CHUNKS
    for c in range(min(_NBUF - 1, _N_CHUNKS)):
      gathers[c] = pltpu.async_copy(
          table_hbm.at[idx_v.at[c]], bufs[c % _NBUF], sgs[c % _NBUF])
    for c in range(_N_CHUNKS):
      gathers[c].wait()
      writes[c] = pltpu.async_copy(bufs[c % _NBUF], out_hbm.at[wid, c],
                                   sws[c % _NBUF])
      nxt = c + _NBUF - 1
      if nxt < _N_CHUNKS:
        if c >= 1:
          writes[c - 1].wait()
        gathers[nxt] = pltpu.async_copy(
            table_hbm.at[idx_v.at[nxt]], bufs[nxt % _NBUF], sgs[nxt % _NBUF])
    for c in range(max(0, _N_CHUNKS - _NBUF), _N_CHUNKS):
      writes[c].wait()

  return k(table, idx)


def kernel(position_ids, pos_emb):
  batch, seq = position_ids.shape
  table = pos_emb.reshape(_TABLE_ROWS, _D)
  idx = position_ids.reshape(_NW, _N_CHUNKS, _CHUNK).astype(jnp.int32)
  out = _gather_sc(table, idx)
  return out.reshape(1, batch, seq, _D)
